# baseline (speedup 1.0000x reference)
import functools

import jax
import jax.numpy as jnp
from jax import lax
from jax.experimental import pallas as pl
from jax.experimental.pallas import tpu as pltpu

IMAGE_H = 96
IMAGE_W = 96
TANFOVX = 0.5
TANFOVY = 0.5

SH_C0 = 0.28209479177387814
SH_C1 = 0.4886025119029199
SH_C2 = [1.0925484305920792, -1.0925484305920792, 0.31539156525252005,
         -1.0925484305920792, 0.5462742152960396]
SH_C3 = [-0.5900435899266435, 2.890611442640554, -0.4570457994644658,
         0.3731763325901154, -0.4570457994644658, 1.445305721320277,
         -0.5900435899266435]

CHUNK = 256
N_BLOCKS = 6
BLOCK_ROWS = IMAGE_H // N_BLOCKS
PIX_BLOCK = BLOCK_ROWS * IMAGE_W
CULL_MARGIN = 1.0
PIECE = 768


def _preprocess_kernel(m3d_t_ref, sh_t_ref, op_ref, sc_t_ref, rot_t_ref,
                       view_ref, proj_ref, campos_ref, compact_ref,
                       counts_ref, *, n_pts, n_pad):
    f32 = jnp.float32
    focal_x = IMAGE_W / (2.0 * TANFOVX)
    focal_y = IMAGE_H / (2.0 * TANFOVY)

    homog_t = jnp.concatenate([m3d_t_ref[...], jnp.ones((1, n_pts), f32)],
                              axis=0)
    t3 = lax.dot_general(view_ref[...], homog_t, (((1,), (0,)), ((), ())),
                         preferred_element_type=f32)
    ph = lax.dot_general(proj_ref[...], homog_t, (((1,), (0,)), ((), ())),
                         preferred_element_type=f32)
    tx = t3[0:1, :]
    ty = t3[1:2, :]
    tz = t3[2:3, :]
    tzc = jnp.where(jnp.abs(tz) < 1e-6, 1e-6, tz)

    p_w = 1.0 / (ph[3:4, :] + 1e-7)
    px = ((ph[0:1, :] * p_w + 1.0) * IMAGE_W - 1.0) * 0.5
    py = ((ph[1:2, :] * p_w + 1.0) * IMAGE_H - 1.0) * 0.5

    qr = rot_t_ref[0:1, :]
    qx = rot_t_ref[1:2, :]
    qy = rot_t_ref[2:3, :]
    qz = rot_t_ref[3:4, :]
    qn = jnp.sqrt(qr * qr + qx * qx + qy * qy + qz * qz) + 1e-12
    qr, qx, qy, qz = qr / qn, qx / qn, qy / qn, qz / qn
    R = ((1.0 - 2.0 * (qy * qy + qz * qz), 2.0 * (qx * qy - qr * qz),
          2.0 * (qx * qz + qr * qy)),
         (2.0 * (qx * qy + qr * qz), 1.0 - 2.0 * (qx * qx + qz * qz),
          2.0 * (qy * qz - qr * qx)),
         (2.0 * (qx * qz - qr * qy), 2.0 * (qy * qz + qr * qx),
          1.0 - 2.0 * (qx * qx + qy * qy)))

    s = tuple(sc_t_ref[j:j + 1, :] for j in range(3))
    M = tuple(tuple(R[a][j] * s[j] for j in range(3)) for a in range(3))
    Sig = tuple(tuple(M[a][0] * M[b][0] + M[a][1] * M[b][1] + M[a][2] * M[b][2]
                      for b in range(3)) for a in range(3))

    limx = 1.3 * TANFOVX
    limy = 1.3 * TANFOVY
    txtz = jnp.clip(tx / tzc, -limx, limx) * tzc
    tytz = jnp.clip(ty / tzc, -limy, limy) * tzc
    inv_tz = 1.0 / tzc
    inv_tz2 = inv_tz * inv_tz
    j00 = focal_x * inv_tz
    j02 = -focal_x * txtz * inv_tz2
    j11 = focal_y * inv_tz
    j12 = -focal_y * tytz * inv_tz2
    W = view_ref
    T0 = tuple(j00 * W[0:1, k:k + 1] + j02 * W[2:3, k:k + 1] for k in range(3))
    T1 = tuple(j11 * W[1:2, k:k + 1] + j12 * W[2:3, k:k + 1] for k in range(3))

    def quad(Ta, Tb):
        u0 = Ta[0] * Sig[0][0] + Ta[1] * Sig[1][0] + Ta[2] * Sig[2][0]
        u1 = Ta[0] * Sig[0][1] + Ta[1] * Sig[1][1] + Ta[2] * Sig[2][1]
        u2 = Ta[0] * Sig[0][2] + Ta[1] * Sig[1][2] + Ta[2] * Sig[2][2]
        return u0 * Tb[0] + u1 * Tb[1] + u2 * Tb[2]

    c00 = quad(T0, T0) + 0.3
    c01 = quad(T0, T1)
    c11 = quad(T1, T1) + 0.3
    det = c00 * c11 - c01 * c01
    det = jnp.where(jnp.abs(det) < 1e-12, 1e-12, det)
    inv_det = 1.0 / det
    ca = c11 * inv_det
    cb = -c01 * inv_det
    cc = c00 * inv_det

    mx = m3d_t_ref[0:1, :]
    my = m3d_t_ref[1:2, :]
    mz = m3d_t_ref[2:3, :]
    dx = mx - campos_ref[0:1, 0:1]
    dy = my - campos_ref[0:1, 1:2]
    dz = mz - campos_ref[0:1, 2:3]
    dn = jnp.sqrt(dx * dx + dy * dy + dz * dz) + 1e-12
    x, y, z = dx / dn, dy / dn, dz / dn
    xx, yy, zz = x * x, y * y, z * z
    xy, yz, xz = x * y, y * z, x * z
    rgb = []
    for c in range(3):
        def shk(k):
            return sh_t_ref[3 * k + c:3 * k + c + 1, :]
        res = SH_C0 * shk(0) - SH_C1 * y * shk(1) + SH_C1 * z * shk(2) - SH_C1 * x * shk(3)
        res = (res + SH_C2[0] * xy * shk(4) + SH_C2[1] * yz * shk(5)
               + SH_C2[2] * (2.0 * zz - xx - yy) * shk(6)
               + SH_C2[3] * xz * shk(7) + SH_C2[4] * (xx - yy) * shk(8))
        res = (res + SH_C3[0] * y * (3.0 * xx - yy) * shk(9)
               + SH_C3[1] * xy * z * shk(10)
               + SH_C3[2] * y * (4.0 * zz - xx - yy) * shk(11)
               + SH_C3[3] * z * (2.0 * zz - 3.0 * xx - 3.0 * yy) * shk(12)
               + SH_C3[4] * x * (4.0 * zz - xx - yy) * shk(13)
               + SH_C3[5] * z * (xx - yy) * shk(14)
               + SH_C3[6] * x * (xx - 3.0 * yy) * shk(15))
        rgb.append(jnp.maximum(res + 0.5, 0.0))

    opm = jnp.where(tz > 0.2, op_ref[0:1, :], 0.0)

    tau = jnp.log(255.0 * opm)
    ry = jnp.sqrt(jnp.maximum(2.0 * tau, 0.0) * c11)
    opflag = 255.0 * opm > 1.0
    flag_rows = []
    for b in range(N_BLOCKS):
        ylo = b * BLOCK_ROWS - CULL_MARGIN
        yhi = (b + 1) * BLOCK_ROWS - 1 + CULL_MARGIN
        flag_rows.append(opflag & (py + ry >= ylo) & (py - ry <= yhi))
    flags = jnp.concatenate([f.astype(f32) for f in flag_rows], axis=0)
    counts_ref[...] = jnp.sum(flags, axis=1, keepdims=True).astype(jnp.int32)

    i_col = lax.broadcasted_iota(jnp.int32, (n_pts, 1), 0)
    j_row = lax.broadcasted_iota(jnp.int32, (1, n_pts), 1)
    eye = (i_col == j_row).astype(f32)
    tz_col = lax.dot_general(eye, tz, (((1,), (1,)), ((), ())),
                             preferred_element_type=f32,
                             precision=lax.Precision.HIGHEST)
    f_cols = lax.dot_general(eye, flags, (((1,), (1,)), ((), ())),
                             preferred_element_type=f32,
                             precision=lax.Precision.HIGHEST)

    before = ((tz < tz_col) | ((tz == tz_col) & (j_row < i_col))).astype(f32)
    rank_blk = lax.dot_general(before, f_cols, (((1,), (0,)), ((), ())),
                               preferred_element_type=f32,
                               precision=lax.Precision.HIGHEST)

    chan = jnp.concatenate(
        [px, py, ca, cb, cc, opm, rgb[0], rgb[1], rgb[2],
         jnp.zeros((7, n_pts), f32)], axis=0)

    pieces = n_pad // PIECE
    s_row = lax.broadcasted_iota(jnp.int32, (1, PIECE), 1)
    counts_i = jnp.sum(flags, axis=1, keepdims=True).astype(jnp.int32)
    for b in range(N_BLOCKS):
        rb = jnp.where(f_cols[:, b:b + 1] > 0.5,
                       rank_blk[:, b:b + 1], -1.0).astype(jnp.int32)
        cnt = counts_i[b, 0]
        for p in range(pieces):
            def _piece(p=p):
                sel = (rb == s_row + p * PIECE).astype(f32)
                compact_ref[b, p * PIECE:(p + 1) * PIECE, :] = lax.dot_general(
                    sel, chan, (((0,), (1,)), ((), ())),
                    preferred_element_type=f32,
                    precision=lax.Precision.HIGHEST)
            if p == 0:
                _piece()
            else:
                pl.when(cnt > p * PIECE)(_piece)


def _composite_kernel(counts_ref, compact_ref, bg_ref, out_ref, *, n_pad):
    f32 = jnp.float32
    K = CHUNK
    N = PIX_BLOCK
    b = pl.program_id(0)
    base = b * N
    n_row = lax.broadcasted_iota(jnp.int32, (1, N), 1) + base
    pixx = (n_row % IMAGE_W).astype(f32)
    pixy = (n_row // IMAGE_W).astype(f32)

    a_iota = lax.broadcasted_iota(jnp.int32, (K, K), 0)
    b_iota = lax.broadcasted_iota(jnp.int32, (K, K), 1)
    ltri = (b_iota < a_iota).astype(f32)

    count = counts_ref[b]
    n_chunks = lax.div(count + (K - 1), K)

    def body(k, state):
        carry, acc = state
        lo = pl.multiple_of(k * K, K)
        px = compact_ref[0, pl.ds(lo, K), 0:1]
        py = compact_ref[0, pl.ds(lo, K), 1:2]
        ca = compact_ref[0, pl.ds(lo, K), 2:3]
        cb = compact_ref[0, pl.ds(lo, K), 3:4]
        cc = compact_ref[0, pl.ds(lo, K), 4:5]
        op = compact_ref[0, pl.ds(lo, K), 5:6]
        rgb = compact_ref[0, pl.ds(lo, K), 6:9]
        dx = pixx - px
        dy = pixy - py
        power = -0.5 * (ca * dx * dx + cc * dy * dy) - cb * dx * dy
        power = jnp.minimum(power, 0.0)
        alpha = jnp.minimum(0.99, op * jnp.exp(power))
        alpha = jnp.where(alpha < 1.0 / 255.0, 0.0, alpha)
        logl = jnp.log(1.0 - alpha)
        s_excl = lax.dot_general(ltri, logl, (((1,), (0,)), ((), ())),
                                 preferred_element_type=f32,
                                 precision=lax.Precision.HIGHEST)
        w = alpha * jnp.exp(carry + s_excl)
        acc = acc + lax.dot_general(rgb, w, (((0,), (0,)), ((), ())),
                                    preferred_element_type=f32)
        carry = carry + s_excl[K - 1:K, :] + logl[K - 1:K, :]
        return carry, acc

    carry = jnp.zeros((1, N), f32)
    acc = jnp.zeros((3, N), f32)
    carry, acc = lax.fori_loop(0, n_chunks, body, (carry, acc))
    acc = acc + bg_ref[...] * jnp.exp(carry)
    out_ref[...] = acc


def kernel(means3D, sh, colors_precomp, opacities, scales, rotations,
           cov3Ds_precomp, bg, viewmatrix, projmatrix, campos):
    f32 = jnp.float32
    P = means3D.shape[0]
    n_pad = ((P + CHUNK - 1) // CHUNK) * CHUNK
    if n_pad == P:
        n_pad = P + CHUNK
    n_pix = IMAGE_H * IMAGE_W

    m3d_t = means3D.T
    sh_t = jnp.transpose(sh, (1, 2, 0)).reshape(48, P)
    op_t = opacities.T
    sc_t = scales.T
    rot_t = rotations.T
    campos2 = campos.reshape(1, 3)
    bg2 = bg.reshape(3, 1)

    compact, counts = pl.pallas_call(
        functools.partial(_preprocess_kernel, n_pts=P, n_pad=n_pad),
        out_shape=(jax.ShapeDtypeStruct((N_BLOCKS, n_pad, 16), f32),
                   jax.ShapeDtypeStruct((N_BLOCKS, 1), jnp.int32)),
    )(m3d_t, sh_t, op_t, sc_t, rot_t, viewmatrix, projmatrix, campos2)

    img = pl.pallas_call(
        functools.partial(_composite_kernel, n_pad=n_pad),
        grid_spec=pltpu.PrefetchScalarGridSpec(
            num_scalar_prefetch=1,
            grid=(N_BLOCKS,),
            in_specs=[
                pl.BlockSpec((1, n_pad, 16), lambda b, cnt: (b, 0, 0)),
                pl.BlockSpec((3, 1), lambda b, cnt: (0, 0)),
            ],
            out_specs=pl.BlockSpec((3, PIX_BLOCK), lambda b, cnt: (0, b)),
        ),
        out_shape=jax.ShapeDtypeStruct((3, n_pix), f32),
    )(counts.reshape(N_BLOCKS), compact, bg2)

    return img.reshape(3, IMAGE_H, IMAGE_W)


# speedup vs baseline: 1.3269x; 1.3269x over previous
import functools

import jax
import jax.numpy as jnp
from jax import lax
from jax.experimental import pallas as pl
from jax.experimental.pallas import tpu as pltpu

IMAGE_H = 96
IMAGE_W = 96
TANFOVX = 0.5
TANFOVY = 0.5

SH_C0 = 0.28209479177387814
SH_C1 = 0.4886025119029199
SH_C2 = [1.0925484305920792, -1.0925484305920792, 0.31539156525252005,
         -1.0925484305920792, 0.5462742152960396]
SH_C3 = [-0.5900435899266435, 2.890611442640554, -0.4570457994644658,
         0.3731763325901154, -0.4570457994644658, 1.445305721320277,
         -0.5900435899266435]

CHUNK = 256
N_BLOCKS = 4
BLOCK_ROWS = IMAGE_H // N_BLOCKS
PIX_BLOCK = BLOCK_ROWS * IMAGE_W
CULL_MARGIN = 1.0
PIECE = 768


def _preprocess_kernel(m3d_t_ref, sh_t_ref, op_ref, sc_t_ref, rot_t_ref,
                       view_ref, proj_ref, campos_ref, compact_ref,
                       counts_ref, *, n_pts, n_pad):
    f32 = jnp.float32
    focal_x = IMAGE_W / (2.0 * TANFOVX)
    focal_y = IMAGE_H / (2.0 * TANFOVY)

    homog_t = jnp.concatenate([m3d_t_ref[...], jnp.ones((1, n_pts), f32)],
                              axis=0)
    t3 = lax.dot_general(view_ref[...], homog_t, (((1,), (0,)), ((), ())),
                         preferred_element_type=f32)
    ph = lax.dot_general(proj_ref[...], homog_t, (((1,), (0,)), ((), ())),
                         preferred_element_type=f32)
    tx = t3[0:1, :]
    ty = t3[1:2, :]
    tz = t3[2:3, :]
    tzc = jnp.where(jnp.abs(tz) < 1e-6, 1e-6, tz)

    p_w = 1.0 / (ph[3:4, :] + 1e-7)
    px = ((ph[0:1, :] * p_w + 1.0) * IMAGE_W - 1.0) * 0.5
    py = ((ph[1:2, :] * p_w + 1.0) * IMAGE_H - 1.0) * 0.5

    qr = rot_t_ref[0:1, :]
    qx = rot_t_ref[1:2, :]
    qy = rot_t_ref[2:3, :]
    qz = rot_t_ref[3:4, :]
    qn = jnp.sqrt(qr * qr + qx * qx + qy * qy + qz * qz) + 1e-12
    qr, qx, qy, qz = qr / qn, qx / qn, qy / qn, qz / qn
    R = ((1.0 - 2.0 * (qy * qy + qz * qz), 2.0 * (qx * qy - qr * qz),
          2.0 * (qx * qz + qr * qy)),
         (2.0 * (qx * qy + qr * qz), 1.0 - 2.0 * (qx * qx + qz * qz),
          2.0 * (qy * qz - qr * qx)),
         (2.0 * (qx * qz - qr * qy), 2.0 * (qy * qz + qr * qx),
          1.0 - 2.0 * (qx * qx + qy * qy)))

    s = tuple(sc_t_ref[j:j + 1, :] for j in range(3))
    M = tuple(tuple(R[a][j] * s[j] for j in range(3)) for a in range(3))
    Sig = tuple(tuple(M[a][0] * M[b][0] + M[a][1] * M[b][1] + M[a][2] * M[b][2]
                      for b in range(3)) for a in range(3))

    limx = 1.3 * TANFOVX
    limy = 1.3 * TANFOVY
    txtz = jnp.clip(tx / tzc, -limx, limx) * tzc
    tytz = jnp.clip(ty / tzc, -limy, limy) * tzc
    inv_tz = 1.0 / tzc
    inv_tz2 = inv_tz * inv_tz
    j00 = focal_x * inv_tz
    j02 = -focal_x * txtz * inv_tz2
    j11 = focal_y * inv_tz
    j12 = -focal_y * tytz * inv_tz2
    W = view_ref
    T0 = tuple(j00 * W[0:1, k:k + 1] + j02 * W[2:3, k:k + 1] for k in range(3))
    T1 = tuple(j11 * W[1:2, k:k + 1] + j12 * W[2:3, k:k + 1] for k in range(3))

    def quad(Ta, Tb):
        u0 = Ta[0] * Sig[0][0] + Ta[1] * Sig[1][0] + Ta[2] * Sig[2][0]
        u1 = Ta[0] * Sig[0][1] + Ta[1] * Sig[1][1] + Ta[2] * Sig[2][1]
        u2 = Ta[0] * Sig[0][2] + Ta[1] * Sig[1][2] + Ta[2] * Sig[2][2]
        return u0 * Tb[0] + u1 * Tb[1] + u2 * Tb[2]

    c00 = quad(T0, T0) + 0.3
    c01 = quad(T0, T1)
    c11 = quad(T1, T1) + 0.3
    det = c00 * c11 - c01 * c01
    det = jnp.where(jnp.abs(det) < 1e-12, 1e-12, det)
    inv_det = 1.0 / det
    ca = c11 * inv_det
    cb = -c01 * inv_det
    cc = c00 * inv_det

    mx = m3d_t_ref[0:1, :]
    my = m3d_t_ref[1:2, :]
    mz = m3d_t_ref[2:3, :]
    dx = mx - campos_ref[0:1, 0:1]
    dy = my - campos_ref[0:1, 1:2]
    dz = mz - campos_ref[0:1, 2:3]
    dn = jnp.sqrt(dx * dx + dy * dy + dz * dz) + 1e-12
    x, y, z = dx / dn, dy / dn, dz / dn
    xx, yy, zz = x * x, y * y, z * z
    xy, yz, xz = x * y, y * z, x * z
    rgb = []
    for c in range(3):
        def shk(k):
            return sh_t_ref[3 * k + c:3 * k + c + 1, :]
        res = SH_C0 * shk(0) - SH_C1 * y * shk(1) + SH_C1 * z * shk(2) - SH_C1 * x * shk(3)
        res = (res + SH_C2[0] * xy * shk(4) + SH_C2[1] * yz * shk(5)
               + SH_C2[2] * (2.0 * zz - xx - yy) * shk(6)
               + SH_C2[3] * xz * shk(7) + SH_C2[4] * (xx - yy) * shk(8))
        res = (res + SH_C3[0] * y * (3.0 * xx - yy) * shk(9)
               + SH_C3[1] * xy * z * shk(10)
               + SH_C3[2] * y * (4.0 * zz - xx - yy) * shk(11)
               + SH_C3[3] * z * (2.0 * zz - 3.0 * xx - 3.0 * yy) * shk(12)
               + SH_C3[4] * x * (4.0 * zz - xx - yy) * shk(13)
               + SH_C3[5] * z * (xx - yy) * shk(14)
               + SH_C3[6] * x * (xx - 3.0 * yy) * shk(15))
        rgb.append(jnp.maximum(res + 0.5, 0.0))

    opm = jnp.where(tz > 0.2, op_ref[0:1, :], 0.0)

    tau = jnp.log(255.0 * opm)
    ry = jnp.sqrt(jnp.maximum(2.0 * tau, 0.0) * c11)
    opflag = 255.0 * opm > 1.0
    flag_rows = []
    for b in range(N_BLOCKS):
        ylo = b * BLOCK_ROWS - CULL_MARGIN
        yhi = (b + 1) * BLOCK_ROWS - 1 + CULL_MARGIN
        flag_rows.append(opflag & (py + ry >= ylo) & (py - ry <= yhi))
    flags = jnp.concatenate([f.astype(f32) for f in flag_rows], axis=0)
    counts_ref[...] = jnp.sum(flags, axis=1, keepdims=True).astype(jnp.int32)

    i_col = lax.broadcasted_iota(jnp.int32, (n_pts, 1), 0)
    j_row = lax.broadcasted_iota(jnp.int32, (1, n_pts), 1)
    eye = (i_col == j_row).astype(f32)
    tzf = jnp.concatenate([tz, flags], axis=0)
    tzf_cols = lax.dot_general(eye, tzf, (((1,), (1,)), ((), ())),
                               preferred_element_type=f32,
                               precision=lax.Precision.HIGHEST)
    tz_col = tzf_cols[:, 0:1]
    f_cols = tzf_cols[:, 1:1 + N_BLOCKS]

    before = ((tz < tz_col) | ((tz == tz_col) & (j_row < i_col))).astype(f32)
    rank_blk = lax.dot_general(before, f_cols, (((1,), (0,)), ((), ())),
                               preferred_element_type=f32)

    chan = jnp.concatenate(
        [px, py, ca, cb, cc, opm, rgb[0], rgb[1], rgb[2],
         jnp.zeros((7, n_pts), f32)], axis=0)

    pieces = n_pad // PIECE
    s_row = lax.broadcasted_iota(jnp.int32, (1, PIECE), 1)
    counts_i = jnp.sum(flags, axis=1, keepdims=True).astype(jnp.int32)
    for b in range(N_BLOCKS):
        rb = jnp.where(f_cols[:, b:b + 1] > 0.5,
                       rank_blk[:, b:b + 1], -1.0).astype(jnp.int32)
        cnt = counts_i[b, 0]
        for p in range(pieces):
            def _piece(p=p):
                sel = (rb == s_row + p * PIECE).astype(f32)
                compact_ref[b, p * PIECE:(p + 1) * PIECE, :] = lax.dot_general(
                    sel, chan, (((0,), (1,)), ((), ())),
                    preferred_element_type=f32,
                    precision=lax.Precision.HIGHEST)
            if p == 0:
                _piece()
            else:
                pl.when(cnt > p * PIECE)(_piece)


def _composite_kernel(counts_ref, compact_ref, bg_ref, out_ref, *, n_pad):
    f32 = jnp.float32
    K = CHUNK
    N = PIX_BLOCK
    b = pl.program_id(0)
    base = b * N
    n_row = lax.broadcasted_iota(jnp.int32, (1, N), 1) + base
    pixx = (n_row % IMAGE_W).astype(f32)
    pixy = (n_row // IMAGE_W).astype(f32)

    a_iota = lax.broadcasted_iota(jnp.int32, (K, K), 0)
    b_iota = lax.broadcasted_iota(jnp.int32, (K, K), 1)
    ltri = (b_iota < a_iota).astype(f32)

    count = counts_ref[b]
    n_chunks = lax.div(count + (K - 1), K)

    def body(k, state):
        carry, acc = state
        lo = pl.multiple_of(k * K, K)
        px = compact_ref[0, pl.ds(lo, K), 0:1]
        py = compact_ref[0, pl.ds(lo, K), 1:2]
        ca = compact_ref[0, pl.ds(lo, K), 2:3]
        cb = compact_ref[0, pl.ds(lo, K), 3:4]
        cc = compact_ref[0, pl.ds(lo, K), 4:5]
        op = compact_ref[0, pl.ds(lo, K), 5:6]
        rgb = compact_ref[0, pl.ds(lo, K), 6:9]
        dx = pixx - px
        dy = pixy - py
        power = -0.5 * (ca * dx * dx + cc * dy * dy) - cb * dx * dy
        power = jnp.minimum(power, 0.0)
        alpha = jnp.minimum(0.99, op * jnp.exp(power))
        alpha = jnp.where(alpha < 1.0 / 255.0, 0.0, alpha)
        logl = jnp.log(1.0 - alpha)
        s_excl = lax.dot_general(ltri, logl, (((1,), (0,)), ((), ())),
                                 preferred_element_type=f32,
                                 precision=lax.Precision.HIGHEST)
        w = alpha * jnp.exp(carry + s_excl)
        acc = acc + lax.dot_general(rgb, w, (((0,), (0,)), ((), ())),
                                    preferred_element_type=f32)
        carry = carry + s_excl[K - 1:K, :] + logl[K - 1:K, :]
        return carry, acc

    carry = jnp.zeros((1, N), f32)
    acc = jnp.zeros((3, N), f32)
    carry, acc = lax.fori_loop(0, n_chunks, body, (carry, acc))
    acc = acc + bg_ref[...] * jnp.exp(carry)
    out_ref[...] = acc


def kernel(means3D, sh, colors_precomp, opacities, scales, rotations,
           cov3Ds_precomp, bg, viewmatrix, projmatrix, campos):
    f32 = jnp.float32
    P = means3D.shape[0]
    n_pad = ((P + CHUNK - 1) // CHUNK) * CHUNK
    if n_pad == P:
        n_pad = P + CHUNK
    n_pix = IMAGE_H * IMAGE_W

    m3d_t = means3D.T
    sh_t = jnp.transpose(sh, (1, 2, 0)).reshape(48, P)
    op_t = opacities.T
    sc_t = scales.T
    rot_t = rotations.T
    campos2 = campos.reshape(1, 3)
    bg2 = bg.reshape(3, 1)

    compact, counts = pl.pallas_call(
        functools.partial(_preprocess_kernel, n_pts=P, n_pad=n_pad),
        out_shape=(jax.ShapeDtypeStruct((N_BLOCKS, n_pad, 16), f32),
                   jax.ShapeDtypeStruct((N_BLOCKS, 1), jnp.int32)),
    )(m3d_t, sh_t, op_t, sc_t, rot_t, viewmatrix, projmatrix, campos2)

    img = pl.pallas_call(
        functools.partial(_composite_kernel, n_pad=n_pad),
        grid_spec=pltpu.PrefetchScalarGridSpec(
            num_scalar_prefetch=1,
            grid=(N_BLOCKS,),
            in_specs=[
                pl.BlockSpec((1, n_pad, 16), lambda b, cnt: (b, 0, 0)),
                pl.BlockSpec((3, 1), lambda b, cnt: (0, 0)),
            ],
            out_specs=pl.BlockSpec((3, PIX_BLOCK), lambda b, cnt: (0, b)),
        ),
        out_shape=jax.ShapeDtypeStruct((3, n_pix), f32),
    )(counts.reshape(N_BLOCKS), compact, bg2)

    return img.reshape(3, IMAGE_H, IMAGE_W)


# speedup vs baseline: 1.4881x; 1.1214x over previous
import functools

import jax
import jax.numpy as jnp
from jax import lax
from jax.experimental import pallas as pl
from jax.experimental.pallas import tpu as pltpu

IMAGE_H = 96
IMAGE_W = 96
TANFOVX = 0.5
TANFOVY = 0.5

SH_C0 = 0.28209479177387814
SH_C1 = 0.4886025119029199
SH_C2 = [1.0925484305920792, -1.0925484305920792, 0.31539156525252005,
         -1.0925484305920792, 0.5462742152960396]
SH_C3 = [-0.5900435899266435, 2.890611442640554, -0.4570457994644658,
         0.3731763325901154, -0.4570457994644658, 1.445305721320277,
         -0.5900435899266435]

CHUNK = 256
N_BLOCKS = 4
BLOCK_ROWS = IMAGE_H // N_BLOCKS
PIX_BLOCK = BLOCK_ROWS * IMAGE_W
CULL_MARGIN = 1.0
PIECE = 768


def _preprocess_kernel(m3d_t_ref, sh_t_ref, op_ref, sc_t_ref, rot_t_ref,
                       view_ref, proj_ref, campos_ref, compact_ref,
                       counts_ref, *, n_pts, n_pad):
    f32 = jnp.float32
    focal_x = IMAGE_W / (2.0 * TANFOVX)
    focal_y = IMAGE_H / (2.0 * TANFOVY)

    homog_t = jnp.concatenate([m3d_t_ref[...], jnp.ones((1, n_pts), f32)],
                              axis=0)
    t3 = lax.dot_general(view_ref[...], homog_t, (((1,), (0,)), ((), ())),
                         preferred_element_type=f32)
    ph = lax.dot_general(proj_ref[...], homog_t, (((1,), (0,)), ((), ())),
                         preferred_element_type=f32)
    tx = t3[0:1, :]
    ty = t3[1:2, :]
    tz = t3[2:3, :]
    tzc = jnp.where(jnp.abs(tz) < 1e-6, 1e-6, tz)

    p_w = 1.0 / (ph[3:4, :] + 1e-7)
    px = ((ph[0:1, :] * p_w + 1.0) * IMAGE_W - 1.0) * 0.5
    py = ((ph[1:2, :] * p_w + 1.0) * IMAGE_H - 1.0) * 0.5

    qr = rot_t_ref[0:1, :]
    qx = rot_t_ref[1:2, :]
    qy = rot_t_ref[2:3, :]
    qz = rot_t_ref[3:4, :]
    qn = jnp.sqrt(qr * qr + qx * qx + qy * qy + qz * qz) + 1e-12
    qr, qx, qy, qz = qr / qn, qx / qn, qy / qn, qz / qn
    R = ((1.0 - 2.0 * (qy * qy + qz * qz), 2.0 * (qx * qy - qr * qz),
          2.0 * (qx * qz + qr * qy)),
         (2.0 * (qx * qy + qr * qz), 1.0 - 2.0 * (qx * qx + qz * qz),
          2.0 * (qy * qz - qr * qx)),
         (2.0 * (qx * qz - qr * qy), 2.0 * (qy * qz + qr * qx),
          1.0 - 2.0 * (qx * qx + qy * qy)))

    s = tuple(sc_t_ref[j:j + 1, :] for j in range(3))
    M = tuple(tuple(R[a][j] * s[j] for j in range(3)) for a in range(3))
    Sig = tuple(tuple(M[a][0] * M[b][0] + M[a][1] * M[b][1] + M[a][2] * M[b][2]
                      for b in range(3)) for a in range(3))

    limx = 1.3 * TANFOVX
    limy = 1.3 * TANFOVY
    txtz = jnp.clip(tx / tzc, -limx, limx) * tzc
    tytz = jnp.clip(ty / tzc, -limy, limy) * tzc
    inv_tz = 1.0 / tzc
    inv_tz2 = inv_tz * inv_tz
    j00 = focal_x * inv_tz
    j02 = -focal_x * txtz * inv_tz2
    j11 = focal_y * inv_tz
    j12 = -focal_y * tytz * inv_tz2
    W = view_ref
    T0 = tuple(j00 * W[0:1, k:k + 1] + j02 * W[2:3, k:k + 1] for k in range(3))
    T1 = tuple(j11 * W[1:2, k:k + 1] + j12 * W[2:3, k:k + 1] for k in range(3))

    def quad(Ta, Tb):
        u0 = Ta[0] * Sig[0][0] + Ta[1] * Sig[1][0] + Ta[2] * Sig[2][0]
        u1 = Ta[0] * Sig[0][1] + Ta[1] * Sig[1][1] + Ta[2] * Sig[2][1]
        u2 = Ta[0] * Sig[0][2] + Ta[1] * Sig[1][2] + Ta[2] * Sig[2][2]
        return u0 * Tb[0] + u1 * Tb[1] + u2 * Tb[2]

    c00 = quad(T0, T0) + 0.3
    c01 = quad(T0, T1)
    c11 = quad(T1, T1) + 0.3
    det = c00 * c11 - c01 * c01
    det = jnp.where(jnp.abs(det) < 1e-12, 1e-12, det)
    inv_det = 1.0 / det
    ca = c11 * inv_det
    cb = -c01 * inv_det
    cc = c00 * inv_det

    mx = m3d_t_ref[0:1, :]
    my = m3d_t_ref[1:2, :]
    mz = m3d_t_ref[2:3, :]
    dx = mx - campos_ref[0:1, 0:1]
    dy = my - campos_ref[0:1, 1:2]
    dz = mz - campos_ref[0:1, 2:3]
    dn = jnp.sqrt(dx * dx + dy * dy + dz * dz) + 1e-12
    x, y, z = dx / dn, dy / dn, dz / dn
    xx, yy, zz = x * x, y * y, z * z
    xy, yz, xz = x * y, y * z, x * z
    rgb = []
    for c in range(3):
        def shk(k):
            return sh_t_ref[3 * k + c:3 * k + c + 1, :]
        res = SH_C0 * shk(0) - SH_C1 * y * shk(1) + SH_C1 * z * shk(2) - SH_C1 * x * shk(3)
        res = (res + SH_C2[0] * xy * shk(4) + SH_C2[1] * yz * shk(5)
               + SH_C2[2] * (2.0 * zz - xx - yy) * shk(6)
               + SH_C2[3] * xz * shk(7) + SH_C2[4] * (xx - yy) * shk(8))
        res = (res + SH_C3[0] * y * (3.0 * xx - yy) * shk(9)
               + SH_C3[1] * xy * z * shk(10)
               + SH_C3[2] * y * (4.0 * zz - xx - yy) * shk(11)
               + SH_C3[3] * z * (2.0 * zz - 3.0 * xx - 3.0 * yy) * shk(12)
               + SH_C3[4] * x * (4.0 * zz - xx - yy) * shk(13)
               + SH_C3[5] * z * (xx - yy) * shk(14)
               + SH_C3[6] * x * (xx - 3.0 * yy) * shk(15))
        rgb.append(jnp.maximum(res + 0.5, 0.0))

    opm = jnp.where(tz > 0.2, op_ref[0:1, :], 0.0)

    tau = jnp.log(255.0 * opm)
    ry = jnp.sqrt(jnp.maximum(2.0 * tau, 0.0) * c11)
    opflag = 255.0 * opm > 1.0
    flag_rows = []
    for b in range(N_BLOCKS):
        ylo = b * BLOCK_ROWS - CULL_MARGIN
        yhi = (b + 1) * BLOCK_ROWS - 1 + CULL_MARGIN
        flag_rows.append(opflag & (py + ry >= ylo) & (py - ry <= yhi))
    flags = jnp.concatenate([f.astype(f32) for f in flag_rows], axis=0)
    counts_ref[...] = jnp.sum(flags, axis=1, keepdims=True).astype(jnp.int32)

    i_col = lax.broadcasted_iota(jnp.int32, (n_pts, 1), 0)
    j_row = lax.broadcasted_iota(jnp.int32, (1, n_pts), 1)
    eye = (i_col == j_row).astype(f32)
    tz_col = tz.reshape(n_pts, 1)
    f_cols = lax.dot_general(eye, flags, (((1,), (1,)), ((), ())),
                             preferred_element_type=f32)

    before = ((tz < tz_col) | ((tz == tz_col) & (j_row < i_col))).astype(f32)
    rank_blk = lax.dot_general(before, f_cols, (((1,), (0,)), ((), ())),
                               preferred_element_type=f32)

    chan = jnp.concatenate(
        [px, py, ca, cb, cc, opm, rgb[0], rgb[1], rgb[2],
         jnp.zeros((7, n_pts), f32)], axis=0)

    pieces = n_pad // PIECE
    s_row = lax.broadcasted_iota(jnp.int32, (1, PIECE), 1)
    counts_i = jnp.sum(flags, axis=1, keepdims=True).astype(jnp.int32)
    for b in range(N_BLOCKS):
        rb = jnp.where(f_cols[:, b:b + 1] > 0.5,
                       rank_blk[:, b:b + 1], -1.0).astype(jnp.int32)
        cnt = counts_i[b, 0]
        for p in range(pieces):
            def _piece(p=p):
                sel = (rb == s_row + p * PIECE).astype(f32)
                compact_ref[b, p * PIECE:(p + 1) * PIECE, :] = lax.dot_general(
                    sel, chan, (((0,), (1,)), ((), ())),
                    preferred_element_type=f32,
                    precision=lax.Precision.HIGHEST)
            if p == 0:
                _piece()
            else:
                pl.when(cnt > p * PIECE)(_piece)


def _composite_kernel(counts_ref, compact_ref, bg_ref, out_ref, *, n_pad):
    f32 = jnp.float32
    K = CHUNK
    N = PIX_BLOCK
    b = pl.program_id(0)
    base = b * N
    n_row = lax.broadcasted_iota(jnp.int32, (1, N), 1) + base
    pixx = (n_row % IMAGE_W).astype(f32)
    pixy = (n_row // IMAGE_W).astype(f32)

    a_iota = lax.broadcasted_iota(jnp.int32, (K, K), 0)
    b_iota = lax.broadcasted_iota(jnp.int32, (K, K), 1)
    ltri = (b_iota < a_iota).astype(f32)

    count = counts_ref[b]
    n_chunks = lax.div(count + (K - 1), K)

    def body(k, state):
        carry, acc = state
        lo = pl.multiple_of(k * K, K)
        px = compact_ref[0, pl.ds(lo, K), 0:1]
        py = compact_ref[0, pl.ds(lo, K), 1:2]
        ca = compact_ref[0, pl.ds(lo, K), 2:3]
        cb = compact_ref[0, pl.ds(lo, K), 3:4]
        cc = compact_ref[0, pl.ds(lo, K), 4:5]
        op = compact_ref[0, pl.ds(lo, K), 5:6]
        rgb = compact_ref[0, pl.ds(lo, K), 6:9]
        dx = pixx - px
        dy = pixy - py
        power = -0.5 * (ca * dx * dx + cc * dy * dy) - cb * dx * dy
        power = jnp.minimum(power, 0.0)
        alpha = jnp.minimum(0.99, op * jnp.exp(power))
        alpha = jnp.where(alpha < 1.0 / 255.0, 0.0, alpha)
        logl = jnp.log(1.0 - alpha)
        s_excl = lax.dot_general(ltri, logl, (((1,), (0,)), ((), ())),
                                 preferred_element_type=f32,
                                 precision=lax.Precision.HIGHEST)
        w = alpha * jnp.exp(carry + s_excl)
        acc = acc + lax.dot_general(rgb, w, (((0,), (0,)), ((), ())),
                                    preferred_element_type=f32)
        carry = carry + s_excl[K - 1:K, :] + logl[K - 1:K, :]
        return carry, acc

    carry = jnp.zeros((1, N), f32)
    acc = jnp.zeros((3, N), f32)
    carry, acc = lax.fori_loop(0, n_chunks, body, (carry, acc))
    acc = acc + bg_ref[...] * jnp.exp(carry)
    out_ref[...] = acc


def kernel(means3D, sh, colors_precomp, opacities, scales, rotations,
           cov3Ds_precomp, bg, viewmatrix, projmatrix, campos):
    f32 = jnp.float32
    P = means3D.shape[0]
    n_pad = ((P + CHUNK - 1) // CHUNK) * CHUNK
    if n_pad == P:
        n_pad = P + CHUNK
    n_pix = IMAGE_H * IMAGE_W

    m3d_t = means3D.T
    sh_t = jnp.transpose(sh, (1, 2, 0)).reshape(48, P)
    op_t = opacities.T
    sc_t = scales.T
    rot_t = rotations.T
    campos2 = campos.reshape(1, 3)
    bg2 = bg.reshape(3, 1)

    compact, counts = pl.pallas_call(
        functools.partial(_preprocess_kernel, n_pts=P, n_pad=n_pad),
        out_shape=(jax.ShapeDtypeStruct((N_BLOCKS, n_pad, 16), f32),
                   jax.ShapeDtypeStruct((N_BLOCKS, 1), jnp.int32)),
    )(m3d_t, sh_t, op_t, sc_t, rot_t, viewmatrix, projmatrix, campos2)

    img = pl.pallas_call(
        functools.partial(_composite_kernel, n_pad=n_pad),
        grid_spec=pltpu.PrefetchScalarGridSpec(
            num_scalar_prefetch=1,
            grid=(N_BLOCKS,),
            in_specs=[
                pl.BlockSpec((1, n_pad, 16), lambda b, cnt: (b, 0, 0)),
                pl.BlockSpec((3, 1), lambda b, cnt: (0, 0)),
            ],
            out_specs=pl.BlockSpec((3, PIX_BLOCK), lambda b, cnt: (0, b)),
        ),
        out_shape=jax.ShapeDtypeStruct((3, n_pix), f32),
    )(counts.reshape(N_BLOCKS), compact, bg2)

    return img.reshape(3, IMAGE_H, IMAGE_W)


# speedup vs baseline: 1.5776x; 1.0602x over previous
import functools

import jax
import jax.numpy as jnp
from jax import lax
from jax.experimental import pallas as pl
from jax.experimental.pallas import tpu as pltpu

IMAGE_H = 96
IMAGE_W = 96
TANFOVX = 0.5
TANFOVY = 0.5

SH_C0 = 0.28209479177387814
SH_C1 = 0.4886025119029199
SH_C2 = [1.0925484305920792, -1.0925484305920792, 0.31539156525252005,
         -1.0925484305920792, 0.5462742152960396]
SH_C3 = [-0.5900435899266435, 2.890611442640554, -0.4570457994644658,
         0.3731763325901154, -0.4570457994644658, 1.445305721320277,
         -0.5900435899266435]

CHUNK = 256
N_BLOCKS = 4
BLOCK_ROWS = IMAGE_H // N_BLOCKS
PIX_BLOCK = BLOCK_ROWS * IMAGE_W
CULL_MARGIN = 1.0
PIECE = 768


def _preprocess_kernel(m3d_t_ref, sh_t_ref, op_ref, sc_t_ref, rot_t_ref,
                       view_ref, proj_ref, campos_ref, compact_ref,
                       counts_ref, *, n_pts, n_pad):
    f32 = jnp.float32
    focal_x = IMAGE_W / (2.0 * TANFOVX)
    focal_y = IMAGE_H / (2.0 * TANFOVY)

    homog_t = jnp.concatenate([m3d_t_ref[...], jnp.ones((1, n_pts), f32)],
                              axis=0)
    t3 = lax.dot_general(view_ref[...], homog_t, (((1,), (0,)), ((), ())),
                         preferred_element_type=f32)
    ph = lax.dot_general(proj_ref[...], homog_t, (((1,), (0,)), ((), ())),
                         preferred_element_type=f32)
    tx = t3[0:1, :]
    ty = t3[1:2, :]
    tz = t3[2:3, :]
    tzc = jnp.where(jnp.abs(tz) < 1e-6, 1e-6, tz)

    p_w = 1.0 / (ph[3:4, :] + 1e-7)
    px = ((ph[0:1, :] * p_w + 1.0) * IMAGE_W - 1.0) * 0.5
    py = ((ph[1:2, :] * p_w + 1.0) * IMAGE_H - 1.0) * 0.5

    qr = rot_t_ref[0:1, :]
    qx = rot_t_ref[1:2, :]
    qy = rot_t_ref[2:3, :]
    qz = rot_t_ref[3:4, :]
    qn = jnp.sqrt(qr * qr + qx * qx + qy * qy + qz * qz) + 1e-12
    qr, qx, qy, qz = qr / qn, qx / qn, qy / qn, qz / qn
    R = ((1.0 - 2.0 * (qy * qy + qz * qz), 2.0 * (qx * qy - qr * qz),
          2.0 * (qx * qz + qr * qy)),
         (2.0 * (qx * qy + qr * qz), 1.0 - 2.0 * (qx * qx + qz * qz),
          2.0 * (qy * qz - qr * qx)),
         (2.0 * (qx * qz - qr * qy), 2.0 * (qy * qz + qr * qx),
          1.0 - 2.0 * (qx * qx + qy * qy)))

    s = tuple(sc_t_ref[j:j + 1, :] for j in range(3))
    M = tuple(tuple(R[a][j] * s[j] for j in range(3)) for a in range(3))
    Sig = tuple(tuple(M[a][0] * M[b][0] + M[a][1] * M[b][1] + M[a][2] * M[b][2]
                      for b in range(3)) for a in range(3))

    limx = 1.3 * TANFOVX
    limy = 1.3 * TANFOVY
    txtz = jnp.clip(tx / tzc, -limx, limx) * tzc
    tytz = jnp.clip(ty / tzc, -limy, limy) * tzc
    inv_tz = 1.0 / tzc
    inv_tz2 = inv_tz * inv_tz
    j00 = focal_x * inv_tz
    j02 = -focal_x * txtz * inv_tz2
    j11 = focal_y * inv_tz
    j12 = -focal_y * tytz * inv_tz2
    W = view_ref
    T0 = tuple(j00 * W[0:1, k:k + 1] + j02 * W[2:3, k:k + 1] for k in range(3))
    T1 = tuple(j11 * W[1:2, k:k + 1] + j12 * W[2:3, k:k + 1] for k in range(3))

    def quad(Ta, Tb):
        u0 = Ta[0] * Sig[0][0] + Ta[1] * Sig[1][0] + Ta[2] * Sig[2][0]
        u1 = Ta[0] * Sig[0][1] + Ta[1] * Sig[1][1] + Ta[2] * Sig[2][1]
        u2 = Ta[0] * Sig[0][2] + Ta[1] * Sig[1][2] + Ta[2] * Sig[2][2]
        return u0 * Tb[0] + u1 * Tb[1] + u2 * Tb[2]

    c00 = quad(T0, T0) + 0.3
    c01 = quad(T0, T1)
    c11 = quad(T1, T1) + 0.3
    det = c00 * c11 - c01 * c01
    det = jnp.where(jnp.abs(det) < 1e-12, 1e-12, det)
    inv_det = 1.0 / det
    ca = c11 * inv_det
    cb = -c01 * inv_det
    cc = c00 * inv_det

    mx = m3d_t_ref[0:1, :]
    my = m3d_t_ref[1:2, :]
    mz = m3d_t_ref[2:3, :]
    dx = mx - campos_ref[0:1, 0:1]
    dy = my - campos_ref[0:1, 1:2]
    dz = mz - campos_ref[0:1, 2:3]
    dn = jnp.sqrt(dx * dx + dy * dy + dz * dz) + 1e-12
    x, y, z = dx / dn, dy / dn, dz / dn
    xx, yy, zz = x * x, y * y, z * z
    xy, yz, xz = x * y, y * z, x * z
    rgb = []
    for c in range(3):
        def shk(k):
            return sh_t_ref[3 * k + c:3 * k + c + 1, :]
        res = SH_C0 * shk(0) - SH_C1 * y * shk(1) + SH_C1 * z * shk(2) - SH_C1 * x * shk(3)
        res = (res + SH_C2[0] * xy * shk(4) + SH_C2[1] * yz * shk(5)
               + SH_C2[2] * (2.0 * zz - xx - yy) * shk(6)
               + SH_C2[3] * xz * shk(7) + SH_C2[4] * (xx - yy) * shk(8))
        res = (res + SH_C3[0] * y * (3.0 * xx - yy) * shk(9)
               + SH_C3[1] * xy * z * shk(10)
               + SH_C3[2] * y * (4.0 * zz - xx - yy) * shk(11)
               + SH_C3[3] * z * (2.0 * zz - 3.0 * xx - 3.0 * yy) * shk(12)
               + SH_C3[4] * x * (4.0 * zz - xx - yy) * shk(13)
               + SH_C3[5] * z * (xx - yy) * shk(14)
               + SH_C3[6] * x * (xx - 3.0 * yy) * shk(15))
        rgb.append(jnp.maximum(res + 0.5, 0.0))

    opm = jnp.where(tz > 0.2, op_ref[0:1, :], 0.0)

    tau = jnp.log(255.0 * opm)
    ry_col = jnp.sqrt(jnp.maximum(2.0 * tau, 0.0) * c11).reshape(n_pts, 1)
    opflag_col = 255.0 * opm.reshape(n_pts, 1) > 1.0
    py_col = py.reshape(n_pts, 1)
    f_col_list = []
    for b in range(N_BLOCKS):
        ylo = b * BLOCK_ROWS - CULL_MARGIN
        yhi = (b + 1) * BLOCK_ROWS - 1 + CULL_MARGIN
        f_col_list.append(opflag_col & (py_col + ry_col >= ylo)
                          & (py_col - ry_col <= yhi))
    f_cols = jnp.concatenate([f.astype(f32) for f in f_col_list], axis=1)
    counts_ref[...] = jnp.sum(f_cols, axis=0, keepdims=True).astype(jnp.int32)

    i_col = lax.broadcasted_iota(jnp.int32, (n_pts, 1), 0)
    j_row = lax.broadcasted_iota(jnp.int32, (1, n_pts), 1)
    tz_col = tz.reshape(n_pts, 1)

    before = ((tz < tz_col) | ((tz == tz_col) & (j_row < i_col))).astype(f32)
    rank_blk = lax.dot_general(before, f_cols, (((1,), (0,)), ((), ())),
                               preferred_element_type=f32)

    chan = jnp.concatenate(
        [px, py, ca, cb, cc, opm, rgb[0], rgb[1], rgb[2],
         jnp.zeros((7, n_pts), f32)], axis=0)

    pieces = n_pad // PIECE
    s_row = lax.broadcasted_iota(jnp.int32, (1, PIECE), 1)
    counts_i = jnp.sum(f_cols, axis=0, keepdims=True).astype(jnp.int32)
    for b in range(N_BLOCKS):
        rb = jnp.where(f_cols[:, b:b + 1] > 0.5,
                       rank_blk[:, b:b + 1], -1.0).astype(jnp.int32)
        cnt = counts_i[0, b]
        for p in range(pieces):
            def _piece(p=p):
                sel = (rb == s_row + p * PIECE).astype(f32)
                compact_ref[b, p * PIECE:(p + 1) * PIECE, :] = lax.dot_general(
                    sel, chan, (((0,), (1,)), ((), ())),
                    preferred_element_type=f32,
                    precision=lax.Precision.HIGHEST)
            if p == 0:
                _piece()
            else:
                pl.when(cnt > p * PIECE)(_piece)


def _composite_kernel(counts_ref, compact_ref, bg_ref, out_ref, *, n_pad):
    f32 = jnp.float32
    K = CHUNK
    N = PIX_BLOCK
    b = pl.program_id(0)
    base = b * N
    n_row = lax.broadcasted_iota(jnp.int32, (1, N), 1) + base
    pixx = (n_row % IMAGE_W).astype(f32)
    pixy = (n_row // IMAGE_W).astype(f32)

    a_iota = lax.broadcasted_iota(jnp.int32, (K, K), 0)
    b_iota = lax.broadcasted_iota(jnp.int32, (K, K), 1)
    ltri = (b_iota < a_iota).astype(f32)

    count = counts_ref[b]
    n_chunks = lax.div(count + (K - 1), K)

    def body(k, state):
        carry, acc = state
        lo = pl.multiple_of(k * K, K)
        px = compact_ref[0, pl.ds(lo, K), 0:1]
        py = compact_ref[0, pl.ds(lo, K), 1:2]
        ca = compact_ref[0, pl.ds(lo, K), 2:3]
        cb = compact_ref[0, pl.ds(lo, K), 3:4]
        cc = compact_ref[0, pl.ds(lo, K), 4:5]
        op = compact_ref[0, pl.ds(lo, K), 5:6]
        rgb = compact_ref[0, pl.ds(lo, K), 6:9]
        dx = pixx - px
        dy = pixy - py
        power = -0.5 * (ca * dx * dx + cc * dy * dy) - cb * dx * dy
        power = jnp.minimum(power, 0.0)
        alpha = jnp.minimum(0.99, op * jnp.exp(power))
        alpha = jnp.where(alpha < 1.0 / 255.0, 0.0, alpha)
        logl = jnp.log(1.0 - alpha)
        s_excl = lax.dot_general(ltri, logl, (((1,), (0,)), ((), ())),
                                 preferred_element_type=f32,
                                 precision=lax.Precision.HIGHEST)
        w = alpha * jnp.exp(carry + s_excl)
        acc = acc + lax.dot_general(rgb, w, (((0,), (0,)), ((), ())),
                                    preferred_element_type=f32)
        carry = carry + s_excl[K - 1:K, :] + logl[K - 1:K, :]
        return carry, acc

    carry = jnp.zeros((1, N), f32)
    acc = jnp.zeros((3, N), f32)
    carry, acc = lax.fori_loop(0, n_chunks, body, (carry, acc))
    acc = acc + bg_ref[...] * jnp.exp(carry)
    out_ref[...] = acc


def kernel(means3D, sh, colors_precomp, opacities, scales, rotations,
           cov3Ds_precomp, bg, viewmatrix, projmatrix, campos):
    f32 = jnp.float32
    P = means3D.shape[0]
    n_pad = ((P + CHUNK - 1) // CHUNK) * CHUNK
    if n_pad == P:
        n_pad = P + CHUNK
    n_pix = IMAGE_H * IMAGE_W

    m3d_t = means3D.T
    sh_t = jnp.transpose(sh, (1, 2, 0)).reshape(48, P)
    op_t = opacities.T
    sc_t = scales.T
    rot_t = rotations.T
    campos2 = campos.reshape(1, 3)
    bg2 = bg.reshape(3, 1)

    compact, counts = pl.pallas_call(
        functools.partial(_preprocess_kernel, n_pts=P, n_pad=n_pad),
        out_shape=(jax.ShapeDtypeStruct((N_BLOCKS, n_pad, 16), f32),
                   jax.ShapeDtypeStruct((1, N_BLOCKS), jnp.int32)),
    )(m3d_t, sh_t, op_t, sc_t, rot_t, viewmatrix, projmatrix, campos2)

    img = pl.pallas_call(
        functools.partial(_composite_kernel, n_pad=n_pad),
        grid_spec=pltpu.PrefetchScalarGridSpec(
            num_scalar_prefetch=1,
            grid=(N_BLOCKS,),
            in_specs=[
                pl.BlockSpec((1, n_pad, 16), lambda b, cnt: (b, 0, 0)),
                pl.BlockSpec((3, 1), lambda b, cnt: (0, 0)),
            ],
            out_specs=pl.BlockSpec((3, PIX_BLOCK), lambda b, cnt: (0, b)),
        ),
        out_shape=jax.ShapeDtypeStruct((3, n_pix), f32),
    )(counts.reshape(N_BLOCKS), compact, bg2)

    return img.reshape(3, IMAGE_H, IMAGE_W)
